# baseline (speedup 1.0000x reference)
import functools

import jax
import jax.numpy as jnp
from jax.experimental import pallas as pl

N_NODES = 10000
N_PAD = 10240
N_EDGES_TOT = 330000
HEADS = 6
ROW_BLOCK = 1280


def _proj_body(h_ref, w_ref, bprev_ref, asrc_w_ref, adst_w_ref,
               xp_ref, asrcT_ref, adstT_ref):
    hb = h_ref[...] + bprev_ref[...]
    xp = jnp.dot(hb, w_ref[...], preferred_element_type=jnp.float32)
    xp_ref[...] = xp
    dout = asrc_w_ref.shape[-1]
    xp3 = xp.reshape(ROW_BLOCK, HEADS, dout)
    asrc = jnp.sum(xp3 * asrc_w_ref[...][None], axis=-1)
    adst = jnp.sum(xp3 * adst_w_ref[...][None], axis=-1)
    asrcT_ref[...] = asrc.T
    adstT_ref[...] = adst.T


def _project(h, W, b_prev, att_src, att_dst):
    n, din = h.shape
    hc = W.shape[1]
    dout = hc // HEADS
    grid = n // ROW_BLOCK
    return pl.pallas_call(
        _proj_body,
        grid=(grid,),
        in_specs=[
            pl.BlockSpec((ROW_BLOCK, din), lambda i: (i, 0)),
            pl.BlockSpec((din, hc), lambda i: (0, 0)),
            pl.BlockSpec((1, din), lambda i: (0, 0)),
            pl.BlockSpec((HEADS, dout), lambda i: (0, 0)),
            pl.BlockSpec((HEADS, dout), lambda i: (0, 0)),
        ],
        out_specs=[
            pl.BlockSpec((ROW_BLOCK, hc), lambda i: (i, 0)),
            pl.BlockSpec((HEADS, ROW_BLOCK), lambda i: (0, i)),
            pl.BlockSpec((HEADS, ROW_BLOCK), lambda i: (0, i)),
        ],
        out_shape=[
            jax.ShapeDtypeStruct((n, hc), jnp.float32),
            jax.ShapeDtypeStruct((HEADS, n), jnp.float32),
            jax.ShapeDtypeStruct((HEADS, n), jnp.float32),
        ],
    )(h, W, b_prev.reshape(1, din), att_src.reshape(HEADS, dout),
      att_dst.reshape(HEADS, dout))


def _edge_phase(xp, asrcT, adstT, src, dst, negative_slope):
    n = xp.shape[0]
    dout = xp.shape[1] // HEADS
    alpha = asrcT[:, src] + adstT[:, dst]
    alpha = jnp.where(alpha >= 0, alpha, negative_slope * alpha)
    m = jax.ops.segment_max(alpha.T, dst, num_segments=n)
    m = jnp.where(jnp.isneginf(m), 0.0, m)
    e = jnp.exp(alpha.T - m[dst])
    s = jax.ops.segment_sum(e, dst, num_segments=n)
    a = e / (s[dst] + 1e-16)
    msg = xp.reshape(n, HEADS, dout)[src] * a[:, :, None]
    out = jax.ops.segment_sum(msg, dst, num_segments=n)
    return out.mean(axis=1)


def kernel(x, edge_index, ptr, W1, att_src1, att_dst1, b1,
           W2, att_src2, att_dst2, b2, W3, att_src3, att_dst3, b3):
    n = x.shape[0]
    loop = jnp.arange(n, dtype=edge_index.dtype)
    src = jnp.concatenate([edge_index[0], loop])
    dst = jnp.concatenate([edge_index[1], loop])
    order = jnp.argsort(dst)
    src_s = src[order]
    dst_s = dst[order]

    zeros_in = jnp.zeros((x.shape[1],), jnp.float32)
    xpad = jnp.pad(x, ((0, N_PAD - n), (0, 0)))
    xp1, asrc1, adst1 = _project(xpad, W1, zeros_in, att_src1, att_dst1)
    h1 = _edge_phase(xp1[:n], asrc1[:, :n], adst1[:, :n], src_s, dst_s, 0.2)
    xp2, asrc2, adst2 = _project(jnp.pad(h1, ((0, N_PAD - n), (0, 0))),
                                 W2, b1, att_src2, att_dst2)
    h2 = _edge_phase(xp2[:n], asrc2[:, :n], adst2[:, :n], src_s, dst_s, 0.2)
    xp3, asrc3, adst3 = _project(jnp.pad(h2, ((0, N_PAD - n), (0, 0))),
                                 W3, b2, att_src3, att_dst3)
    h3 = _edge_phase(xp3[:n], asrc3[:, :n], adst3[:, :n], src_s, dst_s, 0.0)
    final = h3 + b3 + x
    return final[ptr[1:] - 1]


# speedup vs baseline: 8.1063x; 8.1063x over previous
import functools

import jax
import jax.numpy as jnp
from jax import lax
from jax.experimental import pallas as pl
from jax.experimental.pallas import tpu as pltpu
from jax.experimental.pallas import tpu_sc as plsc

N_NODES = 10000
N_PAD = 10240
N_EDGES_TOT = 330000
E_PAD = 332048
RP_PAD = 10248
HEADS = 6
ROW_BLOCK = 1280
NW = 32
NPT = N_PAD // NW
ECH = 1024
XCH = 32
NEG_INF = -3.0e38


def _proj_body(h_ref, w_ref, bprev_ref, asrc_w_ref, adst_w_ref,
               xp_ref, asrc_ref, adst_ref):
    hb = h_ref[...] + bprev_ref[...]
    xp = jnp.dot(hb, w_ref[...], preferred_element_type=jnp.float32)
    xp_ref[...] = xp
    dout = asrc_w_ref.shape[-1]
    xp3 = xp.reshape(ROW_BLOCK, HEADS, dout)
    asrc_ref[...] = jnp.sum(xp3 * asrc_w_ref[...][None], axis=-1)
    adst_ref[...] = jnp.sum(xp3 * adst_w_ref[...][None], axis=-1)


def _project(h, W, b_prev, att_src, att_dst):
    n, din = h.shape
    hc = W.shape[1]
    dout = hc // HEADS
    grid = n // ROW_BLOCK
    return pl.pallas_call(
        _proj_body,
        grid=(grid,),
        in_specs=[
            pl.BlockSpec((ROW_BLOCK, din), lambda i: (i, 0)),
            pl.BlockSpec((din, hc), lambda i: (0, 0)),
            pl.BlockSpec((1, din), lambda i: (0, 0)),
            pl.BlockSpec((HEADS, dout), lambda i: (0, 0)),
            pl.BlockSpec((HEADS, dout), lambda i: (0, 0)),
        ],
        out_specs=[
            pl.BlockSpec((ROW_BLOCK, hc), lambda i: (i, 0)),
            pl.BlockSpec((ROW_BLOCK, HEADS), lambda i: (i, 0)),
            pl.BlockSpec((ROW_BLOCK, HEADS), lambda i: (i, 0)),
        ],
        out_shape=[
            jax.ShapeDtypeStruct((n, hc), jnp.float32),
            jax.ShapeDtypeStruct((n, HEADS), jnp.float32),
            jax.ShapeDtypeStruct((n, HEADS), jnp.float32),
        ],
    )(h, W, b_prev.reshape(1, din), att_src.reshape(HEADS, dout),
      att_dst.reshape(HEADS, dout))


def _i16(x):
    return jnp.full((16,), x, jnp.int32)


def _f16(x):
    return jnp.full((16,), x, jnp.float32)


def _sread(ref, i):
    return jnp.max(plsc.load_gather(ref, [_i16(i)]))


def _gat_edge_body(ns, C, src_hbm, rptr_hbm, asrc_hbm, adst_hbm,
                   xp_hbm, out_hbm, rptr_v, adst_v, idx_v, atab_v,
                   m_v, s_v, is_v, xp_v, acc_v, outrow_v, wsplat_v,
                   sem_g, sem_x):
    wid = lax.axis_index("s") * 2 + lax.axis_index("c")
    v0 = pl.multiple_of(wid * NPT, 8)
    pltpu.sync_copy(rptr_hbm.at[pl.ds(v0, NPT + 8)], rptr_v)
    pltpu.sync_copy(
        adst_hbm.at[pl.ds(pl.multiple_of(wid * (NPT * HEADS), 8),
                          NPT * HEADS)], adst_v)
    pltpu.sync_copy(asrc_hbm, atab_v)
    iota = lax.iota(jnp.int32, 16)

    e0 = _sread(rptr_v, 0)
    e1 = _sread(rptr_v, NPT)
    ws = pl.multiple_of(e0 - lax.rem(e0, 8), 8)
    nch = (e1 - ws + ECH - 1) // ECH

    def _init(i, _):
        m_v[pl.ds(i * 16, 16)] = _f16(NEG_INF)
        s_v[pl.ds(i * 16, 16)] = _f16(0.0)
        return 0
    lax.fori_loop(0, (NPT * HEADS) // 16, _init, 0)
    for h in range(HEADS):
        def _initacc(k, _, h=h):
            acc_v[h, pl.ds(k * 16, 16)] = _f16(0.0)
            return 0
        lax.fori_loop(0, C // 16, _initacc, 0)

    def alpha_group(le, msk, cs, v, h):
        loc = jnp.clip(le - cs, 0, ECH - 1)
        sidx = plsc.load_gather(idx_v, [loc])
        av = plsc.load_gather(atab_v, [sidx * HEADS + h])
        adsplat = plsc.load_gather(adst_v, [_i16(v * HEADS + h)])
        al = av + adsplat
        return jnp.maximum(al, ns * al)

    def sweep_ms(is_sum):
        def chunk_body(c, vc):
            cs = pl.multiple_of(ws + c * ECH, 8)
            ce = cs + ECH
            pltpu.sync_copy(src_hbm.at[pl.ds(cs, ECH)], idx_v)

            def cond(carry):
                v, cont = carry
                return cont & (v < NPT) & (_sread(rptr_v, v) < ce)

            def body(carry):
                v, _ = carry
                rs = _sread(rptr_v, v)
                re = _sread(rptr_v, v + 1)
                a = jnp.maximum(rs, cs)
                b = jnp.minimum(re, ce)
                ng = jnp.maximum(b - a + 15, 0) // 16
                for h in range(HEADS):
                    pos16 = _i16(v * HEADS + h)
                    if is_sum:
                        msplat = plsc.load_gather(m_v, [pos16])

                    def grp(g, acc, h=h, a=a, b=b):
                        base = a + g * 16
                        le = base + iota
                        msk = le < b
                        al = alpha_group(le, msk, cs, v, h)
                        if is_sum:
                            ev = jnp.where(msk, jnp.exp(al - msplat), 0.0)
                            return acc + jnp.sum(ev)
                        al = jnp.where(msk, al, NEG_INF)
                        return jnp.maximum(acc, jnp.max(al))

                    init = 0.0 if is_sum else NEG_INF
                    red = lax.fori_loop(0, ng, grp, init)
                    tgt = s_v if is_sum else m_v
                    old = plsc.load_gather(tgt, [pos16])
                    new = old + red if is_sum else jnp.maximum(old, red)
                    plsc.store_scatter(tgt, [pos16], new, mask=iota < 1)
                done = re <= ce
                return jnp.where(done, v + 1, v), done

            v_out, _ = lax.while_loop(cond, body, (vc, True))
            return v_out
        lax.fori_loop(0, nch, chunk_body, 0)

    sweep_ms(False)
    sweep_ms(True)

    def _inv(i, _):
        sl = pl.ds(i * 16, 16)
        is_v[sl] = 1.0 / (s_v[sl] + 1e-16)
        return 0
    lax.fori_loop(0, (NPT * HEADS) // 16, _inv, 0)

    def chunk3(c, vc):
        cs = pl.multiple_of(ws + c * ECH, 8)
        pltpu.sync_copy(src_hbm.at[pl.ds(cs, ECH)], idx_v)

        def piece(p, vc):
            ps = cs + p * XCH
            pe = ps + XCH
            pltpu.async_copy(
                xp_hbm.at[idx_v.at[pl.ds(pl.multiple_of(p * XCH, 8), XCH)]],
                xp_v, sem_x).wait()

            def cond(carry):
                v, cont = carry
                return cont & (v < NPT) & (_sread(rptr_v, v) < pe)

            def body(carry):
                v, _ = carry
                rs = _sread(rptr_v, v)
                re = _sread(rptr_v, v + 1)
                a = jnp.maximum(rs, ps)
                b = jnp.minimum(re, pe)
                ng = jnp.maximum(b - a + 15, 0) // 16
                for h in range(HEADS):
                    pos16 = _i16(v * HEADS + h)
                    msplat = plsc.load_gather(m_v, [pos16])
                    isplat = plsc.load_gather(is_v, [pos16])

                    def grp(g, _, h=h, a=a, b=b, msplat=msplat,
                            isplat=isplat):
                        base = a + g * 16
                        le = base + iota
                        msk = le < b
                        al = alpha_group(le, msk, cs, v, h)
                        wv = jnp.exp(al - msplat) * isplat
                        wv = jnp.where(msk, wv, 0.0)
                        wsplat_v[...] = wv

                        def edge(l, _, h=h):
                            wspl = plsc.load_gather(wsplat_v, [_i16(l)])
                            eb = jnp.clip(base + l - ps, 0, XCH - 1)

                            def cf(k, _, h=h):
                                xv = xp_v[eb, pl.ds(h * C + k * 16, 16)]
                                plsc.addupdate(
                                    acc_v.at[h, pl.ds(k * 16, 16)],
                                    wspl * xv)
                                return 0
                            lax.fori_loop(0, C // 16, cf, 0)
                            return 0
                        lax.fori_loop(0, 16, edge, 0)
                        return 0
                    lax.fori_loop(0, ng, grp, 0)

                done = re <= pe

                @pl.when(done)
                def _emit():
                    def co(k, _):
                        tot = acc_v[0, pl.ds(k * 16, 16)]
                        for h in range(1, HEADS):
                            tot = tot + acc_v[h, pl.ds(k * 16, 16)]
                        outrow_v[0, pl.ds(k * 16, 16)] = tot * (1.0 / HEADS)
                        for h in range(HEADS):
                            acc_v[h, pl.ds(k * 16, 16)] = _f16(0.0)
                        return 0
                    lax.fori_loop(0, C // 16, co, 0)
                    pltpu.sync_copy(outrow_v, out_hbm.at[pl.ds(v0 + v, 1)])

                return jnp.where(done, v + 1, v), done

            v_out, _ = lax.while_loop(cond, body, (vc, True))
            return v_out

        return lax.fori_loop(0, ECH // XCH, piece, vc)

    lax.fori_loop(0, nch, chunk3, 0)


def _edge_phase_sc(xp, asrc_flat, adst_flat, src_pad, rptr_pad, ns):
    HC = xp.shape[1]
    C = HC // HEADS
    mesh = plsc.VectorSubcoreMesh(core_axis_name="c", subcore_axis_name="s")
    f = pl.kernel(
        functools.partial(_gat_edge_body, ns, C),
        out_type=jax.ShapeDtypeStruct((N_PAD, C), jnp.float32),
        mesh=mesh,
        compiler_params=pltpu.CompilerParams(needs_layout_passes=False),
        scratch_types=[
            pltpu.VMEM((NPT + 8,), jnp.int32),
            pltpu.VMEM((NPT * HEADS,), jnp.float32),
            pltpu.VMEM((ECH,), jnp.int32),
            pltpu.VMEM((N_PAD * HEADS,), jnp.float32),
            pltpu.VMEM((NPT * HEADS,), jnp.float32),
            pltpu.VMEM((NPT * HEADS,), jnp.float32),
            pltpu.VMEM((NPT * HEADS,), jnp.float32),
            pltpu.VMEM((XCH, HC), jnp.float32),
            pltpu.VMEM((HEADS, C), jnp.float32),
            pltpu.VMEM((1, C), jnp.float32),
            pltpu.VMEM((16,), jnp.float32),
            pltpu.SemaphoreType.DMA,
            pltpu.SemaphoreType.DMA,
        ],
    )
    return f(src_pad, rptr_pad, asrc_flat, adst_flat, xp)


def kernel(x, edge_index, ptr, W1, att_src1, att_dst1, b1,
           W2, att_src2, att_dst2, b2, W3, att_src3, att_dst3, b3):
    n = x.shape[0]
    loop = jnp.arange(n, dtype=edge_index.dtype)
    src = jnp.concatenate([edge_index[0], loop])
    dst = jnp.concatenate([edge_index[1], loop])
    key = dst * 32768 + src
    key = jnp.sort(key)
    src_s = key & 32767
    dst_s = key >> 15
    rptr = jnp.searchsorted(dst_s, jnp.arange(RP_PAD, dtype=jnp.int32),
                            side="left").astype(jnp.int32)
    src_pad = jnp.concatenate(
        [src_s, jnp.zeros((E_PAD - N_EDGES_TOT,), jnp.int32)])

    zeros_in = jnp.zeros((x.shape[1],), jnp.float32)
    xpad = jnp.pad(x, ((0, N_PAD - n), (0, 0)))
    xp1, asrc1, adst1 = _project(xpad, W1, zeros_in, att_src1, att_dst1)
    h1 = _edge_phase_sc(xp1, asrc1.reshape(-1), adst1.reshape(-1),
                        src_pad, rptr, 0.2)
    xp2, asrc2, adst2 = _project(h1, W2, b1, att_src2, att_dst2)
    h2 = _edge_phase_sc(xp2, asrc2.reshape(-1), adst2.reshape(-1),
                        src_pad, rptr, 0.2)
    xp3, asrc3, adst3 = _project(h2, W3, b2, att_src3, att_dst3)
    h3 = _edge_phase_sc(xp3, asrc3.reshape(-1), adst3.reshape(-1),
                        src_pad, rptr, 0.0)
    final = h3[:n] + b3 + x
    return final[ptr[1:] - 1]


# speedup vs baseline: 12.0952x; 1.4921x over previous
import functools

import jax
import jax.numpy as jnp
from jax import lax
from jax.experimental import pallas as pl
from jax.experimental.pallas import tpu as pltpu
from jax.experimental.pallas import tpu_sc as plsc

N_NODES = 10000
N_PAD = 10240
N_EDGES_TOT = 330000
E_PAD = 332048
RP_PAD = 10248
HEADS = 6
ROW_BLOCK = 1280
NW = 32
NPT = N_PAD // NW
ECH = 1024
XCH = 32
NEG_INF = -3.0e38


def _proj_body(h_ref, w_ref, bprev_ref, asrc_w_ref, adst_w_ref,
               xp_ref, asrc_ref, adst_ref):
    hb = h_ref[...] + bprev_ref[...]
    xp = jnp.dot(hb, w_ref[...], preferred_element_type=jnp.float32)
    xp_ref[...] = xp
    dout = asrc_w_ref.shape[-1]
    xp3 = xp.reshape(ROW_BLOCK, HEADS, dout)
    asrc_ref[...] = jnp.sum(xp3 * asrc_w_ref[...][None], axis=-1)
    adst_ref[...] = jnp.sum(xp3 * adst_w_ref[...][None], axis=-1)


def _project(h, W, b_prev, att_src, att_dst):
    n, din = h.shape
    hc = W.shape[1]
    dout = hc // HEADS
    grid = n // ROW_BLOCK
    return pl.pallas_call(
        _proj_body,
        grid=(grid,),
        in_specs=[
            pl.BlockSpec((ROW_BLOCK, din), lambda i: (i, 0)),
            pl.BlockSpec((din, hc), lambda i: (0, 0)),
            pl.BlockSpec((1, din), lambda i: (0, 0)),
            pl.BlockSpec((HEADS, dout), lambda i: (0, 0)),
            pl.BlockSpec((HEADS, dout), lambda i: (0, 0)),
        ],
        out_specs=[
            pl.BlockSpec((ROW_BLOCK, hc), lambda i: (i, 0)),
            pl.BlockSpec((ROW_BLOCK, HEADS), lambda i: (i, 0)),
            pl.BlockSpec((ROW_BLOCK, HEADS), lambda i: (i, 0)),
        ],
        out_shape=[
            jax.ShapeDtypeStruct((n, hc), jnp.float32),
            jax.ShapeDtypeStruct((n, HEADS), jnp.float32),
            jax.ShapeDtypeStruct((n, HEADS), jnp.float32),
        ],
    )(h, W, b_prev.reshape(1, din), att_src.reshape(HEADS, dout),
      att_dst.reshape(HEADS, dout))


def _i16(x):
    return jnp.full((16,), x, jnp.int32)


def _f16(x):
    return jnp.full((16,), x, jnp.float32)


def _sread(ref, i):
    return jnp.max(plsc.load_gather(ref, [_i16(i)]))


def _gat_edge_body(ns, C, src_hbm, rptr_hbm, asrc_hbm, adst_hbm,
                   xp_hbm, out_hbm, rptr_v, adst_v, idx_v, atab_v,
                   m_v, s_v, is_v, xp_v, acc_v, outrow_v, wsplat_v,
                   sem_x, sem_y):
    wid = lax.axis_index("s") * 2 + lax.axis_index("c")
    v0 = pl.multiple_of(wid * NPT, 8)
    pltpu.sync_copy(rptr_hbm.at[pl.ds(v0, NPT + 8)], rptr_v)
    pltpu.sync_copy(
        adst_hbm.at[pl.ds(pl.multiple_of(wid * (NPT * HEADS), 8),
                          NPT * HEADS)], adst_v)
    pltpu.sync_copy(asrc_hbm, atab_v)
    iota = lax.iota(jnp.int32, 16)

    e0 = _sread(rptr_v, 0)
    e1 = _sread(rptr_v, NPT)
    ws = pl.multiple_of(e0 - lax.rem(e0, 8), 8)
    nch = (e1 - ws + ECH - 1) // ECH

    def _init(i, _):
        m_v[pl.ds(i * 16, 16)] = _f16(NEG_INF)
        s_v[pl.ds(i * 16, 16)] = _f16(0.0)
        return 0
    lax.fori_loop(0, (NPT * HEADS) // 16, _init, 0)
    for h in range(HEADS):
        def _initacc(k, _, h=h):
            acc_v[h, pl.ds(k * 16, 16)] = _f16(0.0)
            return 0
        lax.fori_loop(0, C // 16, _initacc, 0)

    def alpha_group(le, msk, cs, v, h):
        loc = jnp.clip(le - cs, 0, ECH - 1)
        sidx = plsc.load_gather(idx_v, [loc])
        av = plsc.load_gather(atab_v, [sidx * HEADS + h])
        adsplat = plsc.load_gather(adst_v, [_i16(v * HEADS + h)])
        al = av + adsplat
        return jnp.maximum(al, ns * al)

    def sweep_ms(is_sum):
        def chunk_body(c, vc):
            cs = pl.multiple_of(ws + c * ECH, 8)
            ce = cs + ECH
            pltpu.sync_copy(src_hbm.at[pl.ds(cs, ECH)], idx_v)

            def cond(carry):
                v, cont = carry
                return cont & (v < NPT) & (_sread(rptr_v, v) < ce)

            def body(carry):
                v, _ = carry
                rs = _sread(rptr_v, v)
                re = _sread(rptr_v, v + 1)
                a = jnp.maximum(rs, cs)
                b = jnp.minimum(re, ce)
                ng = jnp.maximum(b - a + 15, 0) // 16
                for h in range(HEADS):
                    pos16 = _i16(v * HEADS + h)
                    if is_sum:
                        msplat = plsc.load_gather(m_v, [pos16])

                    def grp(g, acc, h=h, a=a, b=b):
                        base = a + g * 16
                        le = base + iota
                        msk = le < b
                        al = alpha_group(le, msk, cs, v, h)
                        if is_sum:
                            ev = jnp.where(msk, jnp.exp(al - msplat), 0.0)
                            return acc + jnp.sum(ev)
                        al = jnp.where(msk, al, NEG_INF)
                        return jnp.maximum(acc, jnp.max(al))

                    init = 0.0 if is_sum else NEG_INF
                    red = lax.fori_loop(0, ng, grp, init)
                    tgt = s_v if is_sum else m_v
                    old = plsc.load_gather(tgt, [pos16])
                    new = old + red if is_sum else jnp.maximum(old, red)
                    plsc.store_scatter(tgt, [pos16], new, mask=iota < 1)
                done = re <= ce
                return jnp.where(done, v + 1, v), done

            v_out, _ = lax.while_loop(cond, body, (vc, True))
            return v_out
        lax.fori_loop(0, nch, chunk_body, 0)

    sweep_ms(False)
    sweep_ms(True)

    def _inv(i, _):
        sl = pl.ds(i * 16, 16)
        is_v[sl] = 1.0 / (s_v[sl] + 1e-16)
        return 0
    lax.fori_loop(0, (NPT * HEADS) // 16, _inv, 0)

    XC = XCH if C == 128 else XCH // 2
    npc = ECH // XC

    def chunk3(c, vc):
        cs = pl.multiple_of(ws + c * ECH, 8)
        pltpu.sync_copy(src_hbm.at[pl.ds(cs, ECH)], idx_v)
        cp0 = pltpu.async_copy(
            xp_hbm.at[idx_v.at[pl.ds(0, XC)]],
            xp_v.at[pl.ds(0, XC)], sem_x)
        del cp0

        def piece(p, vc):
            ps = cs + p * XC
            pe = ps + XC
            par = lax.rem(p, 2)
            roff = par * XC

            @pl.when(p + 1 < npc)
            def _prefetch():
                st = pl.multiple_of((p + 1) * XC, 8)

                @pl.when(par == 0)
                def _():
                    pltpu.async_copy(xp_hbm.at[idx_v.at[pl.ds(st, XC)]],
                                     xp_v.at[pl.ds(XC, XC)], sem_y)

                @pl.when(par == 1)
                def _():
                    pltpu.async_copy(xp_hbm.at[idx_v.at[pl.ds(st, XC)]],
                                     xp_v.at[pl.ds(0, XC)], sem_x)

            @pl.when(par == 0)
            def _():
                pltpu.make_async_copy(
                    xp_hbm.at[pl.ds(0, XC)],
                    xp_v.at[pl.ds(0, XC)], sem_x).wait()

            @pl.when(par == 1)
            def _():
                pltpu.make_async_copy(
                    xp_hbm.at[pl.ds(0, XC)],
                    xp_v.at[pl.ds(XC, XC)], sem_y).wait()

            def cond(carry):
                v, cont = carry
                return cont & (v < NPT) & (_sread(rptr_v, v) < pe)

            def body(carry):
                v, _ = carry
                rs = _sread(rptr_v, v)
                re = _sread(rptr_v, v + 1)
                a = jnp.maximum(rs, ps)
                b = jnp.minimum(re, pe)
                ng = jnp.maximum(b - a + 15, 0) // 16
                for h in range(HEADS):
                    pos16 = _i16(v * HEADS + h)
                    msplat = plsc.load_gather(m_v, [pos16])
                    isplat = plsc.load_gather(is_v, [pos16])

                    def grp(g, _, h=h, a=a, b=b, msplat=msplat,
                            isplat=isplat):
                        base = a + g * 16
                        le = base + iota
                        msk = le < b
                        al = alpha_group(le, msk, cs, v, h)
                        wv = jnp.exp(al - msplat) * isplat
                        wv = jnp.where(msk, wv, 0.0)
                        wsplat_v[...] = wv

                        def edge(l, _, h=h):
                            wspl = plsc.load_gather(wsplat_v, [_i16(l)])
                            eb = roff + jnp.clip(base + l - ps, 0, XC - 1)
                            for k in range(C // 16):
                                xv = xp_v[eb, pl.ds(h * C + k * 16, 16)]
                                plsc.addupdate(
                                    acc_v.at[h, pl.ds(k * 16, 16)],
                                    wspl * xv)
                            return 0
                        lax.fori_loop(0, jnp.minimum(b - base, 16), edge, 0)
                        return 0
                    lax.fori_loop(0, ng, grp, 0)

                done = re <= pe

                @pl.when(done)
                def _emit():
                    def co(k, _):
                        tot = acc_v[0, pl.ds(k * 16, 16)]
                        for h in range(1, HEADS):
                            tot = tot + acc_v[h, pl.ds(k * 16, 16)]
                        outrow_v[0, pl.ds(k * 16, 16)] = tot * (1.0 / HEADS)
                        for h in range(HEADS):
                            acc_v[h, pl.ds(k * 16, 16)] = _f16(0.0)
                        return 0
                    lax.fori_loop(0, C // 16, co, 0)
                    pltpu.sync_copy(outrow_v, out_hbm.at[pl.ds(v0 + v, 1)])

                return jnp.where(done, v + 1, v), done

            v_out, _ = lax.while_loop(cond, body, (vc, True))
            return v_out

        return lax.fori_loop(0, npc, piece, vc)

    lax.fori_loop(0, nch, chunk3, 0)


def _edge_phase_sc(xp, asrc_flat, adst_flat, src_pad, rptr_pad, ns):
    HC = xp.shape[1]
    C = HC // HEADS
    mesh = plsc.VectorSubcoreMesh(core_axis_name="c", subcore_axis_name="s")
    f = pl.kernel(
        functools.partial(_gat_edge_body, ns, C),
        out_type=jax.ShapeDtypeStruct((N_PAD, C), jnp.float32),
        mesh=mesh,
        compiler_params=pltpu.CompilerParams(needs_layout_passes=False),
        scratch_types=[
            pltpu.VMEM((NPT + 8,), jnp.int32),
            pltpu.VMEM((NPT * HEADS,), jnp.float32),
            pltpu.VMEM((ECH,), jnp.int32),
            pltpu.VMEM((N_PAD * HEADS,), jnp.float32),
            pltpu.VMEM((NPT * HEADS,), jnp.float32),
            pltpu.VMEM((NPT * HEADS,), jnp.float32),
            pltpu.VMEM((NPT * HEADS,), jnp.float32),
            pltpu.VMEM((2 * (XCH if C == 128 else XCH // 2), HC),
                       jnp.float32),

            pltpu.VMEM((HEADS, C), jnp.float32),
            pltpu.VMEM((1, C), jnp.float32),
            pltpu.VMEM((16,), jnp.float32),
            pltpu.SemaphoreType.DMA,
            pltpu.SemaphoreType.DMA,
        ],
    )
    return f(src_pad, rptr_pad, asrc_flat, adst_flat, xp)


def kernel(x, edge_index, ptr, W1, att_src1, att_dst1, b1,
           W2, att_src2, att_dst2, b2, W3, att_src3, att_dst3, b3):
    n = x.shape[0]
    loop = jnp.arange(n, dtype=edge_index.dtype)
    src = jnp.concatenate([edge_index[0], loop])
    dst = jnp.concatenate([edge_index[1], loop])
    key = dst * 32768 + src
    key = jnp.sort(key)
    src_s = key & 32767
    dst_s = key >> 15
    rptr = jnp.searchsorted(dst_s, jnp.arange(RP_PAD, dtype=jnp.int32),
                            side="left").astype(jnp.int32)
    src_pad = jnp.concatenate(
        [src_s, jnp.zeros((E_PAD - N_EDGES_TOT,), jnp.int32)])

    zeros_in = jnp.zeros((x.shape[1],), jnp.float32)
    xpad = jnp.pad(x, ((0, N_PAD - n), (0, 0)))
    xp1, asrc1, adst1 = _project(xpad, W1, zeros_in, att_src1, att_dst1)
    h1 = _edge_phase_sc(xp1, asrc1.reshape(-1), adst1.reshape(-1),
                        src_pad, rptr, 0.2)
    xp2, asrc2, adst2 = _project(h1, W2, b1, att_src2, att_dst2)
    h2 = _edge_phase_sc(xp2, asrc2.reshape(-1), adst2.reshape(-1),
                        src_pad, rptr, 0.2)
    xp3, asrc3, adst3 = _project(h2, W3, b2, att_src3, att_dst3)
    h3 = _edge_phase_sc(xp3, asrc3.reshape(-1), adst3.reshape(-1),
                        src_pad, rptr, 0.0)
    final = h3[:n] + b3 + x
    return final[ptr[1:] - 1]


# speedup vs baseline: 31.5359x; 2.6073x over previous
import functools

import jax
import jax.numpy as jnp
from jax import lax
from jax.experimental import pallas as pl
from jax.experimental.pallas import tpu as pltpu
from jax.experimental.pallas import tpu_sc as plsc

N_NODES = 10000
N_PAD = 10240
N_EDGES_TOT = 330000
E_PAD = 332048
RP_PAD = 10248
HEADS = 6
ROW_BLOCK = 1280
NW = 32
NPT = N_PAD // NW
ECH = 1024
XCH = 32
NEG_INF = -3.0e38


def _proj_body(h_ref, w_ref, bprev_ref, asrc_w_ref, adst_w_ref,
               xp_ref, asrc_ref, adst_ref):
    hb = h_ref[...] + bprev_ref[...]
    xp = jnp.dot(hb, w_ref[...], preferred_element_type=jnp.float32)
    xp_ref[...] = xp
    dout = asrc_w_ref.shape[-1]
    xp3 = xp.reshape(ROW_BLOCK, HEADS, dout)
    asrc_ref[...] = jnp.sum(xp3 * asrc_w_ref[...][None], axis=-1)
    adst_ref[...] = jnp.sum(xp3 * adst_w_ref[...][None], axis=-1)


def _project(h, W, b_prev, att_src, att_dst):
    n, din = h.shape
    hc = W.shape[1]
    dout = hc // HEADS
    grid = n // ROW_BLOCK
    return pl.pallas_call(
        _proj_body,
        grid=(grid,),
        in_specs=[
            pl.BlockSpec((ROW_BLOCK, din), lambda i: (i, 0)),
            pl.BlockSpec((din, hc), lambda i: (0, 0)),
            pl.BlockSpec((1, din), lambda i: (0, 0)),
            pl.BlockSpec((HEADS, dout), lambda i: (0, 0)),
            pl.BlockSpec((HEADS, dout), lambda i: (0, 0)),
        ],
        out_specs=[
            pl.BlockSpec((ROW_BLOCK, hc), lambda i: (i, 0)),
            pl.BlockSpec((ROW_BLOCK, HEADS), lambda i: (i, 0)),
            pl.BlockSpec((ROW_BLOCK, HEADS), lambda i: (i, 0)),
        ],
        out_shape=[
            jax.ShapeDtypeStruct((n, hc), jnp.float32),
            jax.ShapeDtypeStruct((n, HEADS), jnp.float32),
            jax.ShapeDtypeStruct((n, HEADS), jnp.float32),
        ],
    )(h, W, b_prev.reshape(1, din), att_src.reshape(HEADS, dout),
      att_dst.reshape(HEADS, dout))


def _i16(x):
    return jnp.full((16,), x, jnp.int32)


def _f16(x):
    return jnp.full((16,), x, jnp.float32)


def _sread(ref, i):
    return jnp.max(plsc.load_gather(ref, [_i16(i)]))


def _gat_edge_body(ns, C, src_hbm, rptr_hbm, asrc_hbm, adst_hbm,
                   xp_hbm, out_hbm, rptr_v, adst_v, idx_v, atab_v,
                   m_v, s_v, is_v, xp_v, acc_v, outrow_v, wsplat_v,
                   sem_x, sem_y):
    wid = lax.axis_index("s") * 2 + lax.axis_index("c")
    v0 = pl.multiple_of(wid * NPT, 8)
    pltpu.sync_copy(rptr_hbm.at[pl.ds(v0, NPT + 8)], rptr_v)
    pltpu.sync_copy(
        adst_hbm.at[pl.ds(pl.multiple_of(wid * (NPT * HEADS), 8),
                          NPT * HEADS)], adst_v)
    pltpu.sync_copy(asrc_hbm, atab_v)
    iota = lax.iota(jnp.int32, 16)

    e0 = _sread(rptr_v, 0)
    e1 = _sread(rptr_v, NPT)
    ws = pl.multiple_of(e0 - lax.rem(e0, 8), 8)
    nch = (e1 - ws + ECH - 1) // ECH

    def _init(i, _):
        m_v[pl.ds(i * 16, 16)] = _f16(NEG_INF)
        s_v[pl.ds(i * 16, 16)] = _f16(0.0)
        return 0
    lax.fori_loop(0, (NPT * HEADS) // 16, _init, 0)
    for h in range(HEADS):
        def _initacc(k, _, h=h):
            acc_v[h, pl.ds(k * 16, 16)] = _f16(0.0)
            return 0
        lax.fori_loop(0, C // 16, _initacc, 0)

    def alpha_group(le, msk, cs, v, h):
        loc = jnp.clip(le - cs, 0, ECH - 1)
        sidx = plsc.load_gather(idx_v, [loc])
        av = plsc.load_gather(atab_v, [sidx * HEADS + h])
        adsplat = plsc.load_gather(adst_v, [_i16(v * HEADS + h)])
        al = av + adsplat
        return jnp.maximum(al, ns * al)

    def sweep_ms(is_sum):
        def chunk_body(c, vc):
            cs = pl.multiple_of(ws + c * ECH, 8)
            ce = cs + ECH
            pltpu.sync_copy(src_hbm.at[pl.ds(cs, ECH)], idx_v)

            def cond(carry):
                v, cont = carry
                return cont & (v < NPT) & (_sread(rptr_v, v) < ce)

            def body(carry):
                v, _ = carry
                rs = _sread(rptr_v, v)
                re = _sread(rptr_v, v + 1)
                a = jnp.maximum(rs, cs)
                b = jnp.minimum(re, ce)
                ng = jnp.maximum(b - a + 15, 0) // 16
                for h in range(HEADS):
                    pos16 = _i16(v * HEADS + h)
                    if is_sum:
                        msplat = plsc.load_gather(m_v, [pos16])

                    def grp(g, acc, h=h, a=a, b=b):
                        base = a + g * 16
                        le = base + iota
                        msk = le < b
                        al = alpha_group(le, msk, cs, v, h)
                        if is_sum:
                            ev = jnp.where(msk, jnp.exp(al - msplat), 0.0)
                            return acc + jnp.sum(ev)
                        al = jnp.where(msk, al, NEG_INF)
                        return jnp.maximum(acc, jnp.max(al))

                    init = 0.0 if is_sum else NEG_INF
                    red = lax.fori_loop(0, ng, grp, init)
                    tgt = s_v if is_sum else m_v
                    old = plsc.load_gather(tgt, [pos16])
                    new = old + red if is_sum else jnp.maximum(old, red)
                    plsc.store_scatter(tgt, [pos16], new, mask=iota < 1)
                done = re <= ce
                return jnp.where(done, v + 1, v), done

            v_out, _ = lax.while_loop(cond, body, (vc, True))
            return v_out
        lax.fori_loop(0, nch, chunk_body, 0)

    sweep_ms(False)
    sweep_ms(True)

    def _inv(i, _):
        sl = pl.ds(i * 16, 16)
        is_v[sl] = 1.0 / (s_v[sl] + 1e-16)
        return 0
    lax.fori_loop(0, (NPT * HEADS) // 16, _inv, 0)

    XC = XCH if C == 128 else XCH // 2
    npc = ECH // XC

    def chunk3(c, vc):
        cs = pl.multiple_of(ws + c * ECH, 8)
        pltpu.sync_copy(src_hbm.at[pl.ds(cs, ECH)], idx_v)
        cp0 = pltpu.async_copy(
            xp_hbm.at[idx_v.at[pl.ds(0, XC)]],
            xp_v.at[pl.ds(0, XC)], sem_x)
        del cp0

        def piece(p, vc):
            ps = cs + p * XC
            pe = ps + XC
            par = lax.rem(p, 2)
            roff = par * XC

            @pl.when(p + 1 < npc)
            def _prefetch():
                st = pl.multiple_of((p + 1) * XC, 8)

                @pl.when(par == 0)
                def _():
                    pltpu.async_copy(xp_hbm.at[idx_v.at[pl.ds(st, XC)]],
                                     xp_v.at[pl.ds(XC, XC)], sem_y)

                @pl.when(par == 1)
                def _():
                    pltpu.async_copy(xp_hbm.at[idx_v.at[pl.ds(st, XC)]],
                                     xp_v.at[pl.ds(0, XC)], sem_x)

            @pl.when(par == 0)
            def _():
                pltpu.make_async_copy(
                    xp_hbm.at[pl.ds(0, XC)],
                    xp_v.at[pl.ds(0, XC)], sem_x).wait()

            @pl.when(par == 1)
            def _():
                pltpu.make_async_copy(
                    xp_hbm.at[pl.ds(0, XC)],
                    xp_v.at[pl.ds(XC, XC)], sem_y).wait()

            def cond(carry):
                v, cont = carry
                return cont & (v < NPT) & (_sread(rptr_v, v) < pe)

            def body(carry):
                v, _ = carry
                rs = _sread(rptr_v, v)
                re = _sread(rptr_v, v + 1)
                a = jnp.maximum(rs, ps)
                b = jnp.minimum(re, pe)
                ng = jnp.maximum(b - a + 15, 0) // 16
                for h in range(HEADS):
                    pos16 = _i16(v * HEADS + h)
                    msplat = plsc.load_gather(m_v, [pos16])
                    isplat = plsc.load_gather(is_v, [pos16])

                    def grp(g, _, h=h, a=a, b=b, msplat=msplat,
                            isplat=isplat):
                        base = a + g * 16
                        le = base + iota
                        msk = le < b
                        al = alpha_group(le, msk, cs, v, h)
                        wv = jnp.exp(al - msplat) * isplat
                        wv = jnp.where(msk, wv, 0.0)
                        wsplat_v[...] = wv

                        def edge(l, regs, h=h):
                            wspl = plsc.load_gather(wsplat_v, [_i16(l)])
                            eb = roff + jnp.clip(base + l - ps, 0, XC - 1)
                            return tuple(
                                regs[k] + wspl * xp_v[eb,
                                                      pl.ds(h * C + k * 16,
                                                            16)]
                                for k in range(C // 16))
                        zero = tuple(_f16(0.0) for _ in range(C // 16))
                        regs = lax.fori_loop(
                            0, jnp.minimum(b - base, 16), edge, zero)
                        for k in range(C // 16):
                            plsc.addupdate(acc_v.at[h, pl.ds(k * 16, 16)],
                                           regs[k])
                        return 0
                    lax.fori_loop(0, ng, grp, 0)

                done = re <= pe

                @pl.when(done)
                def _emit():
                    def co(k, _):
                        tot = acc_v[0, pl.ds(k * 16, 16)]
                        for h in range(1, HEADS):
                            tot = tot + acc_v[h, pl.ds(k * 16, 16)]
                        outrow_v[0, pl.ds(k * 16, 16)] = tot * (1.0 / HEADS)
                        for h in range(HEADS):
                            acc_v[h, pl.ds(k * 16, 16)] = _f16(0.0)
                        return 0
                    lax.fori_loop(0, C // 16, co, 0)
                    pltpu.sync_copy(outrow_v, out_hbm.at[pl.ds(v0 + v, 1)])

                return jnp.where(done, v + 1, v), done

            v_out, _ = lax.while_loop(cond, body, (vc, True))
            return v_out

        return lax.fori_loop(0, npc, piece, vc)

    lax.fori_loop(0, nch, chunk3, 0)


def _edge_phase_sc(xp, asrc_flat, adst_flat, src_pad, rptr_pad, ns):
    HC = xp.shape[1]
    C = HC // HEADS
    mesh = plsc.VectorSubcoreMesh(core_axis_name="c", subcore_axis_name="s")
    f = pl.kernel(
        functools.partial(_gat_edge_body, ns, C),
        out_type=jax.ShapeDtypeStruct((N_PAD, C), jnp.float32),
        mesh=mesh,
        compiler_params=pltpu.CompilerParams(needs_layout_passes=False),
        scratch_types=[
            pltpu.VMEM((NPT + 8,), jnp.int32),
            pltpu.VMEM((NPT * HEADS,), jnp.float32),
            pltpu.VMEM((ECH,), jnp.int32),
            pltpu.VMEM((N_PAD * HEADS,), jnp.float32),
            pltpu.VMEM((NPT * HEADS,), jnp.float32),
            pltpu.VMEM((NPT * HEADS,), jnp.float32),
            pltpu.VMEM((NPT * HEADS,), jnp.float32),
            pltpu.VMEM((2 * (XCH if C == 128 else XCH // 2), HC),
                       jnp.float32),

            pltpu.VMEM((HEADS, C), jnp.float32),
            pltpu.VMEM((1, C), jnp.float32),
            pltpu.VMEM((16,), jnp.float32),
            pltpu.SemaphoreType.DMA,
            pltpu.SemaphoreType.DMA,
        ],
    )
    return f(src_pad, rptr_pad, asrc_flat, adst_flat, xp)


def kernel(x, edge_index, ptr, W1, att_src1, att_dst1, b1,
           W2, att_src2, att_dst2, b2, W3, att_src3, att_dst3, b3):
    n = x.shape[0]
    loop = jnp.arange(n, dtype=edge_index.dtype)
    src = jnp.concatenate([edge_index[0], loop])
    dst = jnp.concatenate([edge_index[1], loop])
    key = dst * 32768 + src
    key = jnp.sort(key)
    src_s = key & 32767
    dst_s = key >> 15
    rptr = jnp.searchsorted(dst_s, jnp.arange(RP_PAD, dtype=jnp.int32),
                            side="left").astype(jnp.int32)
    src_pad = jnp.concatenate(
        [src_s, jnp.zeros((E_PAD - N_EDGES_TOT,), jnp.int32)])

    zeros_in = jnp.zeros((x.shape[1],), jnp.float32)
    xpad = jnp.pad(x, ((0, N_PAD - n), (0, 0)))
    xp1, asrc1, adst1 = _project(xpad, W1, zeros_in, att_src1, att_dst1)
    h1 = _edge_phase_sc(xp1, asrc1.reshape(-1), adst1.reshape(-1),
                        src_pad, rptr, 0.2)
    xp2, asrc2, adst2 = _project(h1, W2, b1, att_src2, att_dst2)
    h2 = _edge_phase_sc(xp2, asrc2.reshape(-1), adst2.reshape(-1),
                        src_pad, rptr, 0.2)
    xp3, asrc3, adst3 = _project(h2, W3, b2, att_src3, att_dst3)
    h3 = _edge_phase_sc(xp3, asrc3.reshape(-1), adst3.reshape(-1),
                        src_pad, rptr, 0.0)
    final = h3[:n] + b3 + x
    return final[ptr[1:] - 1]


# speedup vs baseline: 33.0199x; 1.0471x over previous
import functools

import jax
import jax.numpy as jnp
from jax import lax
from jax.experimental import pallas as pl
from jax.experimental.pallas import tpu as pltpu
from jax.experimental.pallas import tpu_sc as plsc

N_NODES = 10000
N_PAD = 10240
N_EDGES_TOT = 330000
E_PAD = 332048
RP_PAD = 10248
HEADS = 6
ROW_BLOCK = 1280
NW = 32
NPT = N_PAD // NW
ECH = 1024
XCH = 32
NEG_INF = -3.0e38


def _proj_body(h_ref, w_ref, bprev_ref, asrc_w_ref, adst_w_ref,
               xp_ref, asrc_ref, adst_ref):
    hb = h_ref[...] + bprev_ref[...]
    xp = jnp.dot(hb, w_ref[...], preferred_element_type=jnp.float32)
    xp_ref[...] = xp
    dout = asrc_w_ref.shape[-1]
    xp3 = xp.reshape(ROW_BLOCK, HEADS, dout)
    asrc_ref[...] = jnp.sum(xp3 * asrc_w_ref[...][None], axis=-1)
    adst_ref[...] = jnp.sum(xp3 * adst_w_ref[...][None], axis=-1)


def _project(h, W, b_prev, att_src, att_dst):
    n, din = h.shape
    hc = W.shape[1]
    dout = hc // HEADS
    grid = n // ROW_BLOCK
    return pl.pallas_call(
        _proj_body,
        grid=(grid,),
        in_specs=[
            pl.BlockSpec((ROW_BLOCK, din), lambda i: (i, 0)),
            pl.BlockSpec((din, hc), lambda i: (0, 0)),
            pl.BlockSpec((1, din), lambda i: (0, 0)),
            pl.BlockSpec((HEADS, dout), lambda i: (0, 0)),
            pl.BlockSpec((HEADS, dout), lambda i: (0, 0)),
        ],
        out_specs=[
            pl.BlockSpec((ROW_BLOCK, hc), lambda i: (i, 0)),
            pl.BlockSpec((ROW_BLOCK, HEADS), lambda i: (i, 0)),
            pl.BlockSpec((ROW_BLOCK, HEADS), lambda i: (i, 0)),
        ],
        out_shape=[
            jax.ShapeDtypeStruct((n, hc), jnp.float32),
            jax.ShapeDtypeStruct((n, HEADS), jnp.float32),
            jax.ShapeDtypeStruct((n, HEADS), jnp.float32),
        ],
    )(h, W, b_prev.reshape(1, din), att_src.reshape(HEADS, dout),
      att_dst.reshape(HEADS, dout))


def _i16(x):
    return jnp.full((16,), x, jnp.int32)


def _f16(x):
    return jnp.full((16,), x, jnp.float32)


def _sread(ref, i):
    return jnp.max(plsc.load_gather(ref, [_i16(i)]))


def _gat_edge_body(ns, C, src_hbm, rptr_hbm, asrc_hbm, adst_hbm,
                   xp_hbm, out_hbm, rptr_v, adst_v, idx_v, atab_v,
                   m_v, s_v, is_v, xp_v, acc_v, outrow_v, wsplat_v,
                   sem_x, sem_y):
    wid = lax.axis_index("s") * 2 + lax.axis_index("c")
    v0 = pl.multiple_of(wid * NPT, 8)
    pltpu.sync_copy(rptr_hbm.at[pl.ds(v0, NPT + 8)], rptr_v)
    pltpu.sync_copy(
        adst_hbm.at[pl.ds(pl.multiple_of(wid * (NPT * HEADS), 8),
                          NPT * HEADS)], adst_v)
    pltpu.sync_copy(asrc_hbm, atab_v)
    iota = lax.iota(jnp.int32, 16)

    e0 = _sread(rptr_v, 0)
    e1 = _sread(rptr_v, NPT)
    ws = pl.multiple_of(e0 - lax.rem(e0, 8), 8)
    nch = (e1 - ws + ECH - 1) // ECH

    def _init(i, _):
        m_v[pl.ds(i * 16, 16)] = _f16(NEG_INF)
        s_v[pl.ds(i * 16, 16)] = _f16(0.0)
        return 0
    lax.fori_loop(0, (NPT * HEADS) // 16, _init, 0)
    for h in range(HEADS):
        def _initacc(k, _, h=h):
            acc_v[h, pl.ds(k * 16, 16)] = _f16(0.0)
            return 0
        lax.fori_loop(0, C // 16, _initacc, 0)

    def alpha_group(le, msk, cs, v, h):
        loc = jnp.clip(le - cs, 0, ECH - 1)
        sidx = plsc.load_gather(idx_v, [loc])
        av = plsc.load_gather(atab_v, [sidx * HEADS + h])
        adsplat = plsc.load_gather(adst_v, [_i16(v * HEADS + h)])
        al = av + adsplat
        return jnp.maximum(al, ns * al)

    def sweep_ms(is_sum):
        def chunk_body(c, carry_c):
            cs = pl.multiple_of(ws + c * ECH, 8)
            ce = cs + ECH
            pltpu.sync_copy(src_hbm.at[pl.ds(cs, ECH)], idx_v)

            def cond(carry):
                v, rs, cont = carry
                return cont & (v < NPT) & (rs < ce)

            def body(carry):
                v, rs, _ = carry
                re = _sread(rptr_v, v + 1)
                a = jnp.maximum(rs, cs)
                b = jnp.minimum(re, ce)
                ng = jnp.maximum(b - a + 15, 0) // 16
                for h in range(HEADS):
                    pos16 = _i16(v * HEADS + h)
                    if is_sum:
                        msplat = plsc.load_gather(m_v, [pos16])

                    def grp(g, acc, h=h, a=a, b=b):
                        base = a + g * 16
                        le = base + iota
                        msk = le < b
                        al = alpha_group(le, msk, cs, v, h)
                        if is_sum:
                            ev = jnp.where(msk, jnp.exp(al - msplat), 0.0)
                            return acc + ev
                        al = jnp.where(msk, al, NEG_INF)
                        return jnp.maximum(acc, al)

                    init = _f16(0.0 if is_sum else NEG_INF)
                    vec = lax.fori_loop(0, ng, grp, init)
                    red = jnp.sum(vec) if is_sum else jnp.max(vec)
                    tgt = s_v if is_sum else m_v
                    old = plsc.load_gather(tgt, [pos16])
                    new = old + red if is_sum else jnp.maximum(old, red)
                    plsc.store_scatter(tgt, [pos16], new, mask=iota < 1)
                done = re <= ce
                return (jnp.where(done, v + 1, v),
                        jnp.where(done, re, rs), done)

            v_out, rs_out, _ = lax.while_loop(
                cond, body, (carry_c[0], carry_c[1], True))
            return (v_out, rs_out)
        lax.fori_loop(0, nch, chunk_body, (0, e0))

    sweep_ms(False)
    sweep_ms(True)

    def _inv(i, _):
        sl = pl.ds(i * 16, 16)
        is_v[sl] = 1.0 / (s_v[sl] + 1e-16)
        return 0
    lax.fori_loop(0, (NPT * HEADS) // 16, _inv, 0)

    XC = XCH if C == 128 else XCH // 2
    npc = ECH // XC

    def chunk3(c, vc):
        cs = pl.multiple_of(ws + c * ECH, 8)
        pltpu.sync_copy(src_hbm.at[pl.ds(cs, ECH)], idx_v)
        cp0 = pltpu.async_copy(
            xp_hbm.at[idx_v.at[pl.ds(0, XC)]],
            xp_v.at[pl.ds(0, XC)], sem_x)
        del cp0

        def piece(p, carry_p):
            ps = cs + p * XC
            pe = ps + XC
            par = lax.rem(p, 2)
            roff = par * XC

            @pl.when(p + 1 < npc)
            def _prefetch():
                st = pl.multiple_of((p + 1) * XC, 8)

                @pl.when(par == 0)
                def _():
                    pltpu.async_copy(xp_hbm.at[idx_v.at[pl.ds(st, XC)]],
                                     xp_v.at[pl.ds(XC, XC)], sem_y)

                @pl.when(par == 1)
                def _():
                    pltpu.async_copy(xp_hbm.at[idx_v.at[pl.ds(st, XC)]],
                                     xp_v.at[pl.ds(0, XC)], sem_x)

            @pl.when(par == 0)
            def _():
                pltpu.make_async_copy(
                    xp_hbm.at[pl.ds(0, XC)],
                    xp_v.at[pl.ds(0, XC)], sem_x).wait()

            @pl.when(par == 1)
            def _():
                pltpu.make_async_copy(
                    xp_hbm.at[pl.ds(0, XC)],
                    xp_v.at[pl.ds(XC, XC)], sem_y).wait()

            def cond(carry):
                v, rs, cont = carry
                return cont & (v < NPT) & (rs < pe)

            def body(carry):
                v, rs, _ = carry
                re = _sread(rptr_v, v + 1)
                a = jnp.maximum(rs, ps)
                b = jnp.minimum(re, pe)
                ng = jnp.maximum(b - a + 15, 0) // 16
                for h in range(HEADS):
                    pos16 = _i16(v * HEADS + h)
                    msplat = plsc.load_gather(m_v, [pos16])
                    isplat = plsc.load_gather(is_v, [pos16])

                    def grp(g, _, h=h, a=a, b=b, msplat=msplat,
                            isplat=isplat):
                        base = a + g * 16
                        le = base + iota
                        msk = le < b
                        al = alpha_group(le, msk, cs, v, h)
                        wv = jnp.exp(al - msplat) * isplat
                        wv = jnp.where(msk, wv, 0.0)
                        wsplat_v[...] = wv

                        def edge(l, regs, h=h):
                            wspl = plsc.load_gather(wsplat_v, [_i16(l)])
                            eb = roff + jnp.clip(base + l - ps, 0, XC - 1)
                            return tuple(
                                regs[k] + wspl * xp_v[eb,
                                                      pl.ds(h * C + k * 16,
                                                            16)]
                                for k in range(C // 16))
                        zero = tuple(_f16(0.0) for _ in range(C // 16))
                        regs = lax.fori_loop(
                            0, jnp.minimum(b - base, 16), edge, zero)
                        for k in range(C // 16):
                            plsc.addupdate(acc_v.at[h, pl.ds(k * 16, 16)],
                                           regs[k])
                        return 0
                    lax.fori_loop(0, ng, grp, 0)

                done = re <= pe

                @pl.when(done)
                def _emit():
                    def co(k, _):
                        tot = acc_v[0, pl.ds(k * 16, 16)]
                        for h in range(1, HEADS):
                            tot = tot + acc_v[h, pl.ds(k * 16, 16)]
                        outrow_v[0, pl.ds(k * 16, 16)] = tot * (1.0 / HEADS)
                        for h in range(HEADS):
                            acc_v[h, pl.ds(k * 16, 16)] = _f16(0.0)
                        return 0
                    lax.fori_loop(0, C // 16, co, 0)
                    pltpu.sync_copy(outrow_v, out_hbm.at[pl.ds(v0 + v, 1)])

                return (jnp.where(done, v + 1, v),
                        jnp.where(done, re, rs), done)

            v_out, rs_out, _ = lax.while_loop(
                cond, body, (carry_p[0], carry_p[1], True))
            return (v_out, rs_out)

        return lax.fori_loop(0, npc, piece, vc)

    lax.fori_loop(0, nch, chunk3, (0, e0))


def _edge_phase_sc(xp, asrc_flat, adst_flat, src_pad, rptr_pad, ns):
    HC = xp.shape[1]
    C = HC // HEADS
    mesh = plsc.VectorSubcoreMesh(core_axis_name="c", subcore_axis_name="s")
    f = pl.kernel(
        functools.partial(_gat_edge_body, ns, C),
        out_type=jax.ShapeDtypeStruct((N_PAD, C), jnp.float32),
        mesh=mesh,
        compiler_params=pltpu.CompilerParams(needs_layout_passes=False),
        scratch_types=[
            pltpu.VMEM((NPT + 8,), jnp.int32),
            pltpu.VMEM((NPT * HEADS,), jnp.float32),
            pltpu.VMEM((ECH,), jnp.int32),
            pltpu.VMEM((N_PAD * HEADS,), jnp.float32),
            pltpu.VMEM((NPT * HEADS,), jnp.float32),
            pltpu.VMEM((NPT * HEADS,), jnp.float32),
            pltpu.VMEM((NPT * HEADS,), jnp.float32),
            pltpu.VMEM((2 * (XCH if C == 128 else XCH // 2), HC),
                       jnp.float32),

            pltpu.VMEM((HEADS, C), jnp.float32),
            pltpu.VMEM((1, C), jnp.float32),
            pltpu.VMEM((16,), jnp.float32),
            pltpu.SemaphoreType.DMA,
            pltpu.SemaphoreType.DMA,
        ],
    )
    return f(src_pad, rptr_pad, asrc_flat, adst_flat, xp)


def kernel(x, edge_index, ptr, W1, att_src1, att_dst1, b1,
           W2, att_src2, att_dst2, b2, W3, att_src3, att_dst3, b3):
    n = x.shape[0]
    loop = jnp.arange(n, dtype=edge_index.dtype)
    src = jnp.concatenate([edge_index[0], loop])
    dst = jnp.concatenate([edge_index[1], loop])
    key = dst * 32768 + src
    key = jnp.sort(key)
    src_s = key & 32767
    dst_s = key >> 15
    rptr = jnp.searchsorted(dst_s, jnp.arange(RP_PAD, dtype=jnp.int32),
                            side="left").astype(jnp.int32)
    src_pad = jnp.concatenate(
        [src_s, jnp.zeros((E_PAD - N_EDGES_TOT,), jnp.int32)])

    zeros_in = jnp.zeros((x.shape[1],), jnp.float32)
    xpad = jnp.pad(x, ((0, N_PAD - n), (0, 0)))
    xp1, asrc1, adst1 = _project(xpad, W1, zeros_in, att_src1, att_dst1)
    h1 = _edge_phase_sc(xp1, asrc1.reshape(-1), adst1.reshape(-1),
                        src_pad, rptr, 0.2)
    xp2, asrc2, adst2 = _project(h1, W2, b1, att_src2, att_dst2)
    h2 = _edge_phase_sc(xp2, asrc2.reshape(-1), adst2.reshape(-1),
                        src_pad, rptr, 0.2)
    xp3, asrc3, adst3 = _project(h2, W3, b2, att_src3, att_dst3)
    h3 = _edge_phase_sc(xp3, asrc3.reshape(-1), adst3.reshape(-1),
                        src_pad, rptr, 0.0)
    final = h3[:n] + b3 + x
    return final[ptr[1:] - 1]
